# baseline (speedup 1.0000x reference)
import functools

import jax
import jax.numpy as jnp
from jax import lax
from jax.experimental import pallas as pl
from jax.experimental.pallas import tpu as pltpu
from jax.experimental.pallas import tpu_sc as plsc

N = 10000
E = 320000
D = 128

NC = 2
NS = 16
NW = NC * NS

NPAD = 10112
RPT = NPAD // NS
CH = 32
CPT = 320
E_PT = CPT * CH
EPAD = NW * E_PT
NSLOT = 8
NPH = 8

_MESH = plsc.VectorSubcoreMesh(core_axis_name="c", subcore_axis_name="s")


NHIST = 10240
CPW = NHIST // NS


@functools.partial(
    pl.kernel,
    out_type=jax.ShapeDtypeStruct((NC * NHIST,), jnp.float32),
    mesh=_MESH,
    compiler_params=pltpu.CompilerParams(needs_layout_passes=False),
    scratch_types=[
        pltpu.VMEM((CPT, CH), jnp.int32),
        pltpu.VMEM((NHIST,), jnp.float32),
        pltpu.VMEM((NS * CPW,), jnp.float32),
        pltpu.VMEM((CPW,), jnp.float32),
        pltpu.VMEM_SHARED((NS * NHIST,), jnp.float32),
        pltpu.SemaphoreType.DMA,
    ],
)
def _sc_degree(edges_hbm, hist_out, idx_v, hist_v, cols_v, acc_v, hist_sh, dsem):
    c = lax.axis_index("c")
    s = lax.axis_index("s")
    wid = c * NS + s
    idx_dma = pltpu.make_async_copy(
        edges_hbm.at[0, pl.ds(wid * CPT, CPT)], idx_v, dsem
    )
    idx_dma.start()

    zeros16 = jnp.zeros((16,), jnp.float32)

    def _zero(i, carry):
        hist_v[pl.ds(i * 16, 16)] = zeros16
        return carry

    lax.fori_loop(0, NHIST // 16, _zero, 0)
    idx_dma.wait()

    ones16 = jnp.ones((16,), jnp.float32)

    def _count(r, carry):
        for j in range(CH // 16):
            ix = idx_v[r, pl.ds(j * 16, 16)]
            plsc.addupdate_scatter(hist_v, [ix], ones16)
        return carry

    lax.fori_loop(0, CPT, _count, 0)

    pltpu.sync_copy(hist_v, hist_sh.at[pl.ds(s * NHIST, NHIST)])
    plsc.subcore_barrier()
    for k in range(NS):
        pltpu.sync_copy(
            hist_sh.at[pl.ds(k * NHIST + s * CPW, CPW)],
            cols_v.at[pl.ds(k * CPW, CPW)],
        )

    def _sum(j, carry):
        val = cols_v[pl.ds(j * 16, 16)]
        for k in range(1, NS):
            val = val + cols_v[pl.ds(k * CPW + j * 16, 16)]
        acc_v[pl.ds(j * 16, 16)] = val
        return carry

    lax.fori_loop(0, CPW // 16, _sum, 0)
    pltpu.sync_copy(acc_v, hist_out.at[pl.ds(c * NHIST + s * CPW, CPW)])


def _prescale_body(x_ref, w_ref, hist_ref, hs_ref):
    deg = hist_ref[:, 0:1] + hist_ref[:, 1:2] + 1.0
    nrm = lax.rsqrt(deg)
    h = jnp.dot(x_ref[...], w_ref[...], preferred_element_type=jnp.float32)
    hs_ref[...] = h * nrm


_PRE_BLK = 1264


def _prescale(x_p, w, hist_t):
    return pl.pallas_call(
        _prescale_body,
        grid=(NPAD // _PRE_BLK,),
        in_specs=[
            pl.BlockSpec((_PRE_BLK, D), lambda i: (i, 0)),
            pl.BlockSpec((D, D), lambda i: (0, 0)),
            pl.BlockSpec((_PRE_BLK, NC), lambda i: (i, 0)),
        ],
        out_specs=pl.BlockSpec((_PRE_BLK, D), lambda i: (i, 0)),
        out_shape=jax.ShapeDtypeStruct((NPAD, D), jnp.float32),
    )(x_p, w, hist_t)


@functools.partial(
    pl.kernel,
    out_type=jax.ShapeDtypeStruct((NC, NPAD, D), jnp.float32),
    mesh=_MESH,
    compiler_params=pltpu.CompilerParams(needs_layout_passes=False),
    scratch_types=[
        pltpu.VMEM((CPT // NPH, CH), jnp.int32),
        pltpu.VMEM((CPT // NPH, CH), jnp.int32),
        [pltpu.VMEM((CH, D), jnp.float32)] * NSLOT,
        pltpu.VMEM_SHARED((NPAD, D), jnp.float32),
        [pltpu.SemaphoreType.DMA] * NSLOT,
        [pltpu.SemaphoreType.DMA] * NSLOT,
        pltpu.SemaphoreType.DMA,
    ],
)
def _sc_aggregate(
    hs_hbm, edges_hbm, out_hbm, sidx, didx, rows, agg_sh, gsems, ssems, seedsem
):
    c = lax.axis_index("c")
    s = lax.axis_index("s")
    wid = c * NS + s
    base = s * RPT

    @pl.when(c == 0)
    def _():
        pltpu.async_copy(
            hs_hbm.at[pl.ds(base, RPT)], agg_sh.at[pl.ds(base, RPT)], seedsem
        )

    @pl.when(c != 0)
    def _():
        for k in range(5):
            pltpu.async_copy(
                hs_hbm.at[pl.ds(N, 112)],
                agg_sh.at[pl.ds(base + k * 112, 112)],
                seedsem,
            )
        pltpu.async_copy(
            hs_hbm.at[pl.ds(N, 72)], agg_sh.at[pl.ds(base + 560, 72)], seedsem
        )

    CPP = CPT // NPH
    for p in range(NPH):
        pltpu.sync_copy(edges_hbm.at[0, pl.ds(wid * CPT + p * CPP, CPP)], sidx)
        pltpu.sync_copy(edges_hbm.at[1, pl.ds(wid * CPT + p * CPP, CPP)], didx)

        for t in range(NSLOT):
            pltpu.async_copy(hs_hbm.at[sidx.at[t]], rows[t], gsems[t])

        if p == 0:
            @pl.when(c == 0)
            def _():
                pltpu.make_async_copy(
                    hs_hbm.at[pl.ds(base, RPT)],
                    agg_sh.at[pl.ds(base, RPT)],
                    seedsem,
                ).wait()

            @pl.when(c != 0)
            def _():
                for k in range(5):
                    pltpu.make_async_copy(
                        hs_hbm.at[pl.ds(N, 112)],
                        agg_sh.at[pl.ds(base + k * 112, 112)],
                        seedsem,
                    ).wait()
                pltpu.make_async_copy(
                    hs_hbm.at[pl.ds(N, 72)],
                    agg_sh.at[pl.ds(base + 560, 72)],
                    seedsem,
                ).wait()

            plsc.subcore_barrier()

        def _group(i, carry):
            for t in range(NSLOT):
                j = i * NSLOT + t
                pltpu.make_async_copy(
                    hs_hbm.at[sidx.at[j]], rows[t], gsems[t]
                ).wait()
                pltpu.async_copy(rows[t], agg_sh.at[didx.at[j]], ssems[t], add=True)

                @pl.when(i < CPP // NSLOT - 1)
                def _():
                    pltpu.make_async_copy(
                        rows[t], agg_sh.at[didx.at[j]], ssems[t]
                    ).wait()
                    pltpu.async_copy(
                        hs_hbm.at[sidx.at[j + NSLOT]], rows[t], gsems[t]
                    )

            return carry

        lax.fori_loop(0, CPP // NSLOT, _group, 0)

        for t in range(NSLOT):
            pltpu.make_async_copy(
                rows[t], agg_sh.at[didx.at[CPP - NSLOT + t]], ssems[t]
            ).wait()

    plsc.subcore_barrier()
    pltpu.sync_copy(
        agg_sh.at[pl.ds(base, RPT)], out_hbm.at[c, pl.ds(base, RPT)]
    )


def _final_body(a_ref, b_ref, out_ref):
    r = a_ref[0] + b_ref[0]
    ss = jnp.sum(r * r, axis=1, keepdims=True)
    denom = jnp.maximum(jnp.sqrt(ss), 1e-12)
    out_ref[...] = jnp.tanh(r / denom)


_FIN_BLK = 2000


def _final(partials):
    return pl.pallas_call(
        _final_body,
        grid=(N // _FIN_BLK,),
        in_specs=[
            pl.BlockSpec((1, _FIN_BLK, D), lambda i: (0, i, 0)),
            pl.BlockSpec((1, _FIN_BLK, D), lambda i: (1, i, 0)),
        ],
        out_specs=pl.BlockSpec((_FIN_BLK, D), lambda i: (i, 0)),
        out_shape=jax.ShapeDtypeStruct((N, D), jnp.float32),
    )(partials, partials)


@jax.jit
def kernel(x, edge_index, W):
    npe = EPAD - E
    pad_src = N + jnp.arange(npe, dtype=jnp.int32) % (NPAD - N)
    pad_dst = jnp.arange(npe, dtype=jnp.int32) % N
    pad2 = jnp.stack([pad_src, pad_dst]).reshape(2, npe // CH, CH)
    edges_p = jnp.concatenate(
        [edge_index.reshape(2, E // CH, CH), pad2], axis=1
    )

    x_p = jnp.zeros((NPAD, D), jnp.float32).at[:N].set(x)

    hist = _sc_degree(edges_p)
    hist_t = hist.reshape(NC, NHIST).T
    hs = _prescale(x_p, W, hist_t)
    partials = _sc_aggregate(hs, edges_p)
    return _final(partials)


# speedup vs baseline: 1.1534x; 1.1534x over previous
import functools

import jax
import jax.numpy as jnp
from jax import lax
from jax.experimental import pallas as pl
from jax.experimental.pallas import tpu as pltpu
from jax.experimental.pallas import tpu_sc as plsc

N = 10000
E = 320000
D = 128

NC = 2
NS = 16
NW = NC * NS

NPAD = 10112
RPT = NPAD // NS
CH = 64
CPT = 160
E_PT = CPT * CH
EPAD = NW * E_PT
NSLOT = 4
NPH = 4

_MESH = plsc.VectorSubcoreMesh(core_axis_name="c", subcore_axis_name="s")


NHIST = 10240
CPW = NHIST // NS


@functools.partial(
    pl.kernel,
    out_type=jax.ShapeDtypeStruct((NC * NHIST,), jnp.float32),
    mesh=_MESH,
    compiler_params=pltpu.CompilerParams(needs_layout_passes=False),
    scratch_types=[
        pltpu.VMEM((CPT, CH), jnp.int32),
        pltpu.VMEM((NHIST,), jnp.float32),
        pltpu.VMEM((NS * CPW,), jnp.float32),
        pltpu.VMEM((CPW,), jnp.float32),
        pltpu.VMEM_SHARED((NS * NHIST,), jnp.float32),
        pltpu.SemaphoreType.DMA,
    ],
)
def _sc_degree(edges_hbm, hist_out, idx_v, hist_v, cols_v, acc_v, hist_sh, dsem):
    c = lax.axis_index("c")
    s = lax.axis_index("s")
    wid = c * NS + s
    idx_dma = pltpu.make_async_copy(
        edges_hbm.at[0, pl.ds(wid * CPT, CPT)], idx_v, dsem
    )
    idx_dma.start()

    zeros16 = jnp.zeros((16,), jnp.float32)

    def _zero(i, carry):
        hist_v[pl.ds(i * 16, 16)] = zeros16
        return carry

    lax.fori_loop(0, NHIST // 16, _zero, 0)
    idx_dma.wait()

    ones16 = jnp.ones((16,), jnp.float32)

    def _count(r, carry):
        for j in range(CH // 16):
            ix = idx_v[r, pl.ds(j * 16, 16)]
            plsc.addupdate_scatter(hist_v, [ix], ones16)
        return carry

    lax.fori_loop(0, CPT, _count, 0)

    pltpu.sync_copy(hist_v, hist_sh.at[pl.ds(s * NHIST, NHIST)])
    plsc.subcore_barrier()
    for k in range(NS):
        pltpu.sync_copy(
            hist_sh.at[pl.ds(k * NHIST + s * CPW, CPW)],
            cols_v.at[pl.ds(k * CPW, CPW)],
        )

    def _sum(j, carry):
        val = cols_v[pl.ds(j * 16, 16)]
        for k in range(1, NS):
            val = val + cols_v[pl.ds(k * CPW + j * 16, 16)]
        acc_v[pl.ds(j * 16, 16)] = val
        return carry

    lax.fori_loop(0, CPW // 16, _sum, 0)
    pltpu.sync_copy(acc_v, hist_out.at[pl.ds(c * NHIST + s * CPW, CPW)])


def _prescale_body(x_ref, w_ref, hist_ref, hs_ref):
    deg = hist_ref[:, 0:1] + hist_ref[:, 1:2] + 1.0
    nrm = lax.rsqrt(deg)
    h = jnp.dot(x_ref[...], w_ref[...], preferred_element_type=jnp.float32)
    hs_ref[...] = h * nrm


_PRE_BLK = 1264


def _prescale(x_p, w, hist_t):
    return pl.pallas_call(
        _prescale_body,
        grid=(NPAD // _PRE_BLK,),
        in_specs=[
            pl.BlockSpec((_PRE_BLK, D), lambda i: (i, 0)),
            pl.BlockSpec((D, D), lambda i: (0, 0)),
            pl.BlockSpec((_PRE_BLK, NC), lambda i: (i, 0)),
        ],
        out_specs=pl.BlockSpec((_PRE_BLK, D), lambda i: (i, 0)),
        out_shape=jax.ShapeDtypeStruct((NPAD, D), jnp.float32),
    )(x_p, w, hist_t)


@functools.partial(
    pl.kernel,
    out_type=jax.ShapeDtypeStruct((NC, NPAD, D), jnp.float32),
    mesh=_MESH,
    compiler_params=pltpu.CompilerParams(needs_layout_passes=False),
    scratch_types=[
        pltpu.VMEM((CPT // NPH, CH), jnp.int32),
        pltpu.VMEM((CPT // NPH, CH), jnp.int32),
        [pltpu.VMEM((CH, D), jnp.float32)] * NSLOT,
        pltpu.VMEM_SHARED((NPAD, D), jnp.float32),
        [pltpu.SemaphoreType.DMA] * NSLOT,
        [pltpu.SemaphoreType.DMA] * NSLOT,
        pltpu.SemaphoreType.DMA,
    ],
)
def _sc_aggregate(
    hs_hbm, edges_hbm, out_hbm, sidx, didx, rows, agg_sh, gsems, ssems, seedsem
):
    c = lax.axis_index("c")
    s = lax.axis_index("s")
    wid = c * NS + s
    base = s * RPT

    @pl.when(c == 0)
    def _():
        pltpu.async_copy(
            hs_hbm.at[pl.ds(base, RPT)], agg_sh.at[pl.ds(base, RPT)], seedsem
        )

    @pl.when(c != 0)
    def _():
        for k in range(5):
            pltpu.async_copy(
                hs_hbm.at[pl.ds(N, 112)],
                agg_sh.at[pl.ds(base + k * 112, 112)],
                seedsem,
            )
        pltpu.async_copy(
            hs_hbm.at[pl.ds(N, 72)], agg_sh.at[pl.ds(base + 560, 72)], seedsem
        )

    CPP = CPT // NPH
    for p in range(NPH):
        pltpu.sync_copy(edges_hbm.at[0, pl.ds(wid * CPT + p * CPP, CPP)], sidx)
        pltpu.sync_copy(edges_hbm.at[1, pl.ds(wid * CPT + p * CPP, CPP)], didx)

        for t in range(NSLOT):
            pltpu.async_copy(hs_hbm.at[sidx.at[t]], rows[t], gsems[t])

        if p == 0:
            @pl.when(c == 0)
            def _():
                pltpu.make_async_copy(
                    hs_hbm.at[pl.ds(base, RPT)],
                    agg_sh.at[pl.ds(base, RPT)],
                    seedsem,
                ).wait()

            @pl.when(c != 0)
            def _():
                for k in range(5):
                    pltpu.make_async_copy(
                        hs_hbm.at[pl.ds(N, 112)],
                        agg_sh.at[pl.ds(base + k * 112, 112)],
                        seedsem,
                    ).wait()
                pltpu.make_async_copy(
                    hs_hbm.at[pl.ds(N, 72)],
                    agg_sh.at[pl.ds(base + 560, 72)],
                    seedsem,
                ).wait()

            plsc.subcore_barrier()

        def _group(i, carry):
            for t in range(NSLOT):
                j = i * NSLOT + t
                pltpu.make_async_copy(
                    hs_hbm.at[sidx.at[j]], rows[t], gsems[t]
                ).wait()
                pltpu.async_copy(rows[t], agg_sh.at[didx.at[j]], ssems[t], add=True)

                @pl.when(i < CPP // NSLOT - 1)
                def _():
                    pltpu.make_async_copy(
                        rows[t], agg_sh.at[didx.at[j]], ssems[t]
                    ).wait()
                    pltpu.async_copy(
                        hs_hbm.at[sidx.at[j + NSLOT]], rows[t], gsems[t]
                    )

            return carry

        lax.fori_loop(0, CPP // NSLOT, _group, 0)

        for t in range(NSLOT):
            pltpu.make_async_copy(
                rows[t], agg_sh.at[didx.at[CPP - NSLOT + t]], ssems[t]
            ).wait()

    plsc.subcore_barrier()
    pltpu.sync_copy(
        agg_sh.at[pl.ds(base, RPT)], out_hbm.at[c, pl.ds(base, RPT)]
    )


def _final_body(a_ref, b_ref, out_ref):
    r = a_ref[0] + b_ref[0]
    ss = jnp.sum(r * r, axis=1, keepdims=True)
    denom = jnp.maximum(jnp.sqrt(ss), 1e-12)
    out_ref[...] = jnp.tanh(r / denom)


_FIN_BLK = 2000


def _final(partials):
    return pl.pallas_call(
        _final_body,
        grid=(N // _FIN_BLK,),
        in_specs=[
            pl.BlockSpec((1, _FIN_BLK, D), lambda i: (0, i, 0)),
            pl.BlockSpec((1, _FIN_BLK, D), lambda i: (1, i, 0)),
        ],
        out_specs=pl.BlockSpec((_FIN_BLK, D), lambda i: (i, 0)),
        out_shape=jax.ShapeDtypeStruct((N, D), jnp.float32),
    )(partials, partials)


@jax.jit
def kernel(x, edge_index, W):
    npe = EPAD - E
    pad_src = N + jnp.arange(npe, dtype=jnp.int32) % (NPAD - N)
    pad_dst = jnp.arange(npe, dtype=jnp.int32) % N
    pad2 = jnp.stack([pad_src, pad_dst]).reshape(2, npe // CH, CH)
    edges_p = jnp.concatenate(
        [edge_index.reshape(2, E // CH, CH), pad2], axis=1
    )

    x_p = jnp.zeros((NPAD, D), jnp.float32).at[:N].set(x)

    hist = _sc_degree(edges_p)
    hist_t = hist.reshape(NC, NHIST).T
    hs = _prescale(x_p, W, hist_t)
    partials = _sc_aggregate(hs, edges_p)
    return _final(partials)


# speedup vs baseline: 1.1688x; 1.0134x over previous
import functools

import jax
import jax.numpy as jnp
from jax import lax
from jax.experimental import pallas as pl
from jax.experimental.pallas import tpu as pltpu
from jax.experimental.pallas import tpu_sc as plsc

N = 10000
E = 320000
D = 128

NC = 2
NS = 16
NW = NC * NS

NPAD = 10112
RPT = NPAD // NS
CH = 64
CPT = 160
E_PT = CPT * CH
EPAD = NW * E_PT
NSLOT = 4
NPH = 4

_MESH = plsc.VectorSubcoreMesh(core_axis_name="c", subcore_axis_name="s")


NHIST = 10240
CPW = NHIST // NS


@functools.partial(
    pl.kernel,
    out_type=jax.ShapeDtypeStruct((NC * NHIST,), jnp.float32),
    mesh=_MESH,
    compiler_params=pltpu.CompilerParams(needs_layout_passes=False),
    scratch_types=[
        pltpu.VMEM((CPT, CH), jnp.int32),
        pltpu.VMEM((NHIST,), jnp.float32),
        pltpu.VMEM((NS * CPW,), jnp.float32),
        pltpu.VMEM((CPW,), jnp.float32),
        pltpu.VMEM_SHARED((NS * NHIST,), jnp.float32),
        pltpu.SemaphoreType.DMA,
    ],
)
def _sc_degree(edges_hbm, hist_out, idx_v, hist_v, cols_v, acc_v, hist_sh, dsem):
    c = lax.axis_index("c")
    s = lax.axis_index("s")
    wid = c * NS + s
    idx_dma = pltpu.make_async_copy(
        edges_hbm.at[0, pl.ds(wid * CPT, CPT)], idx_v, dsem
    )
    idx_dma.start()

    zeros16 = jnp.zeros((16,), jnp.float32)

    def _zero(i, carry):
        hist_v[pl.ds(i * 16, 16)] = zeros16
        return carry

    lax.fori_loop(0, NHIST // 16, _zero, 0)
    idx_dma.wait()

    ones16 = jnp.ones((16,), jnp.float32)

    def _count(r, carry):
        for j in range(CH // 16):
            ix = idx_v[r, pl.ds(j * 16, 16)]
            plsc.addupdate_scatter(hist_v, [ix], ones16)
        return carry

    lax.fori_loop(0, CPT, _count, 0)

    pltpu.sync_copy(hist_v, hist_sh.at[pl.ds(s * NHIST, NHIST)])
    plsc.subcore_barrier()
    for k in range(NS):
        pltpu.sync_copy(
            hist_sh.at[pl.ds(k * NHIST + s * CPW, CPW)],
            cols_v.at[pl.ds(k * CPW, CPW)],
        )

    def _sum(j, carry):
        val = cols_v[pl.ds(j * 16, 16)]
        for k in range(1, NS):
            val = val + cols_v[pl.ds(k * CPW + j * 16, 16)]
        acc_v[pl.ds(j * 16, 16)] = val
        return carry

    lax.fori_loop(0, CPW // 16, _sum, 0)
    pltpu.sync_copy(acc_v, hist_out.at[pl.ds(c * NHIST + s * CPW, CPW)])


def _prescale_body(x_ref, w_ref, hist_ref, hs_ref):
    row = pl.program_id(0) * _PRE_BLK + lax.broadcasted_iota(
        jnp.int32, (_PRE_BLK, 1), 0
    )
    deg = hist_ref[:, 0:1] + hist_ref[:, 1:2] + 1.0
    nrm = lax.rsqrt(deg)
    h = jnp.dot(x_ref[...], w_ref[...], preferred_element_type=jnp.float32)
    hs_ref[...] = jnp.where(row < N, h * nrm, 0.0)


_PRE_BLK = 1264


def _prescale(x, w, hist_t):
    return pl.pallas_call(
        _prescale_body,
        grid=(NPAD // _PRE_BLK,),
        in_specs=[
            pl.BlockSpec((_PRE_BLK, D), lambda i: (i, 0)),
            pl.BlockSpec((D, D), lambda i: (0, 0)),
            pl.BlockSpec((_PRE_BLK, NC), lambda i: (i, 0)),
        ],
        out_specs=pl.BlockSpec((_PRE_BLK, D), lambda i: (i, 0)),
        out_shape=jax.ShapeDtypeStruct((NPAD, D), jnp.float32),
    )(x, w, hist_t)


@functools.partial(
    pl.kernel,
    out_type=jax.ShapeDtypeStruct((NC, NPAD, D), jnp.float32),
    mesh=_MESH,
    compiler_params=pltpu.CompilerParams(needs_layout_passes=False),
    scratch_types=[
        pltpu.VMEM((CPT // NPH, CH), jnp.int32),
        pltpu.VMEM((CPT // NPH, CH), jnp.int32),
        [pltpu.VMEM((CH, D), jnp.float32)] * NSLOT,
        pltpu.VMEM_SHARED((NPAD, D), jnp.float32),
        [pltpu.SemaphoreType.DMA] * NSLOT,
        [pltpu.SemaphoreType.DMA] * NSLOT,
        pltpu.SemaphoreType.DMA,
    ],
)
def _sc_aggregate(
    hs_hbm, edges_hbm, out_hbm, sidx, didx, rows, agg_sh, gsems, ssems, seedsem
):
    c = lax.axis_index("c")
    s = lax.axis_index("s")
    wid = c * NS + s
    base = s * RPT

    @pl.when(c == 0)
    def _():
        pltpu.async_copy(
            hs_hbm.at[pl.ds(base, RPT)], agg_sh.at[pl.ds(base, RPT)], seedsem
        )

    @pl.when(c != 0)
    def _():
        for k in range(5):
            pltpu.async_copy(
                hs_hbm.at[pl.ds(N, 112)],
                agg_sh.at[pl.ds(base + k * 112, 112)],
                seedsem,
            )
        pltpu.async_copy(
            hs_hbm.at[pl.ds(N, 72)], agg_sh.at[pl.ds(base + 560, 72)], seedsem
        )

    CPP = CPT // NPH
    for p in range(NPH):
        pltpu.sync_copy(edges_hbm.at[0, pl.ds(wid * CPT + p * CPP, CPP)], sidx)
        pltpu.sync_copy(edges_hbm.at[1, pl.ds(wid * CPT + p * CPP, CPP)], didx)

        for t in range(NSLOT):
            pltpu.async_copy(hs_hbm.at[sidx.at[t]], rows[t], gsems[t])

        if p == 0:
            @pl.when(c == 0)
            def _():
                pltpu.make_async_copy(
                    hs_hbm.at[pl.ds(base, RPT)],
                    agg_sh.at[pl.ds(base, RPT)],
                    seedsem,
                ).wait()

            @pl.when(c != 0)
            def _():
                for k in range(5):
                    pltpu.make_async_copy(
                        hs_hbm.at[pl.ds(N, 112)],
                        agg_sh.at[pl.ds(base + k * 112, 112)],
                        seedsem,
                    ).wait()
                pltpu.make_async_copy(
                    hs_hbm.at[pl.ds(N, 72)],
                    agg_sh.at[pl.ds(base + 560, 72)],
                    seedsem,
                ).wait()

            plsc.subcore_barrier()

        def _group(i, carry):
            for t in range(NSLOT):
                j = i * NSLOT + t
                pltpu.make_async_copy(
                    hs_hbm.at[sidx.at[j]], rows[t], gsems[t]
                ).wait()
                pltpu.async_copy(rows[t], agg_sh.at[didx.at[j]], ssems[t], add=True)

                @pl.when(i < CPP // NSLOT - 1)
                def _():
                    pltpu.make_async_copy(
                        rows[t], agg_sh.at[didx.at[j]], ssems[t]
                    ).wait()
                    pltpu.async_copy(
                        hs_hbm.at[sidx.at[j + NSLOT]], rows[t], gsems[t]
                    )

            return carry

        lax.fori_loop(0, CPP // NSLOT, _group, 0)

        for t in range(NSLOT):
            pltpu.make_async_copy(
                rows[t], agg_sh.at[didx.at[CPP - NSLOT + t]], ssems[t]
            ).wait()

    plsc.subcore_barrier()
    pltpu.sync_copy(
        agg_sh.at[pl.ds(base, RPT)], out_hbm.at[c, pl.ds(base, RPT)]
    )


def _final_body(a_ref, b_ref, out_ref):
    r = a_ref[0] + b_ref[0]
    ss = jnp.sum(r * r, axis=1, keepdims=True)
    denom = jnp.maximum(jnp.sqrt(ss), 1e-12)
    out_ref[...] = jnp.tanh(r / denom)


_FIN_BLK = 2000


def _final(partials):
    return pl.pallas_call(
        _final_body,
        grid=(N // _FIN_BLK,),
        in_specs=[
            pl.BlockSpec((1, _FIN_BLK, D), lambda i: (0, i, 0)),
            pl.BlockSpec((1, _FIN_BLK, D), lambda i: (1, i, 0)),
        ],
        out_specs=pl.BlockSpec((_FIN_BLK, D), lambda i: (i, 0)),
        out_shape=jax.ShapeDtypeStruct((N, D), jnp.float32),
    )(partials, partials)


@jax.jit
def kernel(x, edge_index, W):
    npe = EPAD - E
    pad_src = N + jnp.arange(npe, dtype=jnp.int32) % (NPAD - N)
    pad_dst = jnp.arange(npe, dtype=jnp.int32) % N
    pad2 = jnp.stack([pad_src, pad_dst]).reshape(2, npe // CH, CH)
    edges_p = jnp.concatenate(
        [edge_index.reshape(2, E // CH, CH), pad2], axis=1
    )

    hist = _sc_degree(edges_p)
    hist_t = hist.reshape(NC, NHIST).T
    hs = _prescale(x, W, hist_t)
    partials = _sc_aggregate(hs, edges_p)
    return _final(partials)


# speedup vs baseline: 1.1918x; 1.0197x over previous
import functools

import jax
import jax.numpy as jnp
from jax import lax
from jax.experimental import pallas as pl
from jax.experimental.pallas import tpu as pltpu
from jax.experimental.pallas import tpu_sc as plsc

N = 10000
E = 320000
D = 128

NC = 2
NS = 16
NW = NC * NS

NPAD = 10112
RPT = NPAD // NS
CH = 64
CPT = 160
E_PT = CPT * CH
EPAD = NW * E_PT
NSLOT = 4
NPH = 4

_MESH = plsc.VectorSubcoreMesh(core_axis_name="c", subcore_axis_name="s")


NHIST = 10240
CPW = NHIST // NS
E_DEG = 9984
E_LAST = E - 31 * E_DEG


@functools.partial(
    pl.kernel,
    out_type=jax.ShapeDtypeStruct((NC * NHIST,), jnp.float32),
    mesh=_MESH,
    compiler_params=pltpu.CompilerParams(needs_layout_passes=False),
    scratch_types=[
        pltpu.VMEM((2, E_LAST), jnp.int32),
        pltpu.VMEM((NHIST,), jnp.float32),
        pltpu.VMEM((NS * CPW,), jnp.float32),
        pltpu.VMEM((CPW,), jnp.float32),
        pltpu.VMEM_SHARED((NS * NHIST,), jnp.float32),
        pltpu.SemaphoreType.DMA,
    ],
)
def _sc_degree(ei_hbm, hist_out, idx_v, hist_v, cols_v, acc_v, hist_sh, dsem):
    c = lax.axis_index("c")
    s = lax.axis_index("s")
    wid = c * NS + s
    last = wid == NW - 1

    @pl.when(jnp.logical_not(last))
    def _():
        pltpu.async_copy(
            ei_hbm.at[:, pl.ds(wid * E_DEG, E_DEG)],
            idx_v.at[:, pl.ds(0, E_DEG)],
            dsem,
        )

    @pl.when(last)
    def _():
        pltpu.async_copy(
            ei_hbm.at[:, pl.ds((NW - 1) * E_DEG, E_LAST)], idx_v, dsem
        )

    zeros16 = jnp.zeros((16,), jnp.float32)

    def _zero(i, carry):
        hist_v[pl.ds(i * 16, 16)] = zeros16
        return carry

    lax.fori_loop(0, NHIST // 16, _zero, 0)

    @pl.when(jnp.logical_not(last))
    def _():
        pltpu.make_async_copy(
            ei_hbm.at[:, pl.ds(wid * E_DEG, E_DEG)],
            idx_v.at[:, pl.ds(0, E_DEG)],
            dsem,
        ).wait()

    @pl.when(last)
    def _():
        pltpu.make_async_copy(
            ei_hbm.at[:, pl.ds((NW - 1) * E_DEG, E_LAST)], idx_v, dsem
        ).wait()

    ones16 = jnp.ones((16,), jnp.float32)

    def _count(j, carry):
        ix = idx_v[0, pl.ds(j * 16, 16)]
        plsc.addupdate_scatter(hist_v, [ix], ones16)
        return carry

    nj = jnp.where(last, E_LAST // 16, E_DEG // 16)
    lax.fori_loop(0, nj, _count, 0)

    pltpu.sync_copy(hist_v, hist_sh.at[pl.ds(s * NHIST, NHIST)])
    plsc.subcore_barrier()
    for k in range(NS):
        pltpu.sync_copy(
            hist_sh.at[pl.ds(k * NHIST + s * CPW, CPW)],
            cols_v.at[pl.ds(k * CPW, CPW)],
        )

    def _sum(j, carry):
        val = cols_v[pl.ds(j * 16, 16)]
        for k in range(1, NS):
            val = val + cols_v[pl.ds(k * CPW + j * 16, 16)]
        acc_v[pl.ds(j * 16, 16)] = val
        return carry

    lax.fori_loop(0, CPW // 16, _sum, 0)
    pltpu.sync_copy(acc_v, hist_out.at[pl.ds(c * NHIST + s * CPW, CPW)])


def _prescale_body(x_ref, w_ref, hist_ref, hs_ref):
    row = pl.program_id(0) * _PRE_BLK + lax.broadcasted_iota(
        jnp.int32, (_PRE_BLK, 1), 0
    )
    deg = hist_ref[:, 0:1] + hist_ref[:, 1:2] + 1.0
    nrm = lax.rsqrt(deg)
    h = jnp.dot(x_ref[...], w_ref[...], preferred_element_type=jnp.float32)
    hs_ref[...] = jnp.where(row < N, h * nrm, 0.0)


_PRE_BLK = 1264


def _prescale(x, w, hist_t):
    return pl.pallas_call(
        _prescale_body,
        grid=(NPAD // _PRE_BLK,),
        in_specs=[
            pl.BlockSpec((_PRE_BLK, D), lambda i: (i, 0)),
            pl.BlockSpec((D, D), lambda i: (0, 0)),
            pl.BlockSpec((_PRE_BLK, NC), lambda i: (i, 0)),
        ],
        out_specs=pl.BlockSpec((_PRE_BLK, D), lambda i: (i, 0)),
        out_shape=jax.ShapeDtypeStruct((NPAD, D), jnp.float32),
    )(x, w, hist_t)


@functools.partial(
    pl.kernel,
    out_type=jax.ShapeDtypeStruct((NC, NPAD, D), jnp.float32),
    mesh=_MESH,
    compiler_params=pltpu.CompilerParams(needs_layout_passes=False),
    scratch_types=[
        pltpu.VMEM((CPT // NPH, CH), jnp.int32),
        pltpu.VMEM((CPT // NPH, CH), jnp.int32),
        [pltpu.VMEM((CH, D), jnp.float32)] * NSLOT,
        pltpu.VMEM_SHARED((NPAD, D), jnp.float32),
        [pltpu.SemaphoreType.DMA] * NSLOT,
        [pltpu.SemaphoreType.DMA] * NSLOT,
        pltpu.SemaphoreType.DMA,
    ],
)
def _sc_aggregate(
    hs_hbm, edges_hbm, out_hbm, sidx, didx, rows, agg_sh, gsems, ssems, seedsem
):
    c = lax.axis_index("c")
    s = lax.axis_index("s")
    wid = c * NS + s
    base = s * RPT

    @pl.when(c == 0)
    def _():
        pltpu.async_copy(
            hs_hbm.at[pl.ds(base, RPT)], agg_sh.at[pl.ds(base, RPT)], seedsem
        )

    @pl.when(c != 0)
    def _():
        for k in range(5):
            pltpu.async_copy(
                hs_hbm.at[pl.ds(N, 112)],
                agg_sh.at[pl.ds(base + k * 112, 112)],
                seedsem,
            )
        pltpu.async_copy(
            hs_hbm.at[pl.ds(N, 72)], agg_sh.at[pl.ds(base + 560, 72)], seedsem
        )

    CPP = CPT // NPH
    for p in range(NPH):
        pltpu.sync_copy(edges_hbm.at[0, pl.ds(wid * CPT + p * CPP, CPP)], sidx)
        pltpu.sync_copy(edges_hbm.at[1, pl.ds(wid * CPT + p * CPP, CPP)], didx)

        for t in range(NSLOT):
            pltpu.async_copy(hs_hbm.at[sidx.at[t]], rows[t], gsems[t])

        if p == 0:
            @pl.when(c == 0)
            def _():
                pltpu.make_async_copy(
                    hs_hbm.at[pl.ds(base, RPT)],
                    agg_sh.at[pl.ds(base, RPT)],
                    seedsem,
                ).wait()

            @pl.when(c != 0)
            def _():
                for k in range(5):
                    pltpu.make_async_copy(
                        hs_hbm.at[pl.ds(N, 112)],
                        agg_sh.at[pl.ds(base + k * 112, 112)],
                        seedsem,
                    ).wait()
                pltpu.make_async_copy(
                    hs_hbm.at[pl.ds(N, 72)],
                    agg_sh.at[pl.ds(base + 560, 72)],
                    seedsem,
                ).wait()

            plsc.subcore_barrier()

        def _group(i, carry):
            for t in range(NSLOT):
                j = i * NSLOT + t
                pltpu.make_async_copy(
                    hs_hbm.at[sidx.at[j]], rows[t], gsems[t]
                ).wait()
                pltpu.async_copy(rows[t], agg_sh.at[didx.at[j]], ssems[t], add=True)

                @pl.when(i < CPP // NSLOT - 1)
                def _():
                    pltpu.make_async_copy(
                        rows[t], agg_sh.at[didx.at[j]], ssems[t]
                    ).wait()
                    pltpu.async_copy(
                        hs_hbm.at[sidx.at[j + NSLOT]], rows[t], gsems[t]
                    )

            return carry

        lax.fori_loop(0, CPP // NSLOT, _group, 0)

        for t in range(NSLOT):
            pltpu.make_async_copy(
                rows[t], agg_sh.at[didx.at[CPP - NSLOT + t]], ssems[t]
            ).wait()

    plsc.subcore_barrier()
    pltpu.sync_copy(
        agg_sh.at[pl.ds(base, RPT)], out_hbm.at[c, pl.ds(base, RPT)]
    )


def _final_body(a_ref, b_ref, out_ref):
    r = a_ref[0] + b_ref[0]
    ss = jnp.sum(r * r, axis=1, keepdims=True)
    denom = jnp.maximum(jnp.sqrt(ss), 1e-12)
    out_ref[...] = jnp.tanh(r / denom)


_FIN_BLK = 2000


def _final(partials):
    return pl.pallas_call(
        _final_body,
        grid=(N // _FIN_BLK,),
        in_specs=[
            pl.BlockSpec((1, _FIN_BLK, D), lambda i: (0, i, 0)),
            pl.BlockSpec((1, _FIN_BLK, D), lambda i: (1, i, 0)),
        ],
        out_specs=pl.BlockSpec((_FIN_BLK, D), lambda i: (i, 0)),
        out_shape=jax.ShapeDtypeStruct((N, D), jnp.float32),
    )(partials, partials)


@jax.jit
def kernel(x, edge_index, W):
    npe = EPAD - E
    pad_src = N + jnp.arange(npe, dtype=jnp.int32) % (NPAD - N)
    pad_dst = jnp.arange(npe, dtype=jnp.int32) % N
    pad2 = jnp.stack([pad_src, pad_dst]).reshape(2, npe // CH, CH)
    edges_p = jnp.concatenate(
        [edge_index.reshape(2, E // CH, CH), pad2], axis=1
    )

    hist = _sc_degree(edge_index)
    hist_t = hist.reshape(NC, NHIST).T
    hs = _prescale(x, W, hist_t)
    partials = _sc_aggregate(hs, edges_p)
    return _final(partials)


# speedup vs baseline: 1.2283x; 1.0306x over previous
import functools

import jax
import jax.numpy as jnp
from jax import lax
from jax.experimental import pallas as pl
from jax.experimental.pallas import tpu as pltpu
from jax.experimental.pallas import tpu_sc as plsc

N = 10000
E = 320000
D = 128

NC = 2
NS = 16
NW = NC * NS

NPAD = 10112
RPT = NPAD // NS
CH = 64
CPT = 160
E_PT = CPT * CH
EPAD = NW * E_PT
NSLOT = 4
NPH = 4

_MESH = plsc.VectorSubcoreMesh(core_axis_name="c", subcore_axis_name="s")


NHIST = 10240
CPW = NHIST // NS
E_DEG = 9984
E_LAST = E - 31 * E_DEG


@functools.partial(
    pl.kernel,
    out_type=jax.ShapeDtypeStruct((NC * NHIST,), jnp.float32),
    mesh=_MESH,
    compiler_params=pltpu.CompilerParams(needs_layout_passes=False),
    scratch_types=[
        pltpu.VMEM((2, E_LAST), jnp.int32),
        pltpu.VMEM((NHIST,), jnp.float32),
        pltpu.VMEM((NS * CPW,), jnp.float32),
        pltpu.VMEM((CPW,), jnp.float32),
        pltpu.VMEM_SHARED((NS * NHIST,), jnp.float32),
        pltpu.SemaphoreType.DMA,
    ],
)
def _sc_degree(ei_hbm, hist_out, idx_v, hist_v, cols_v, acc_v, hist_sh, dsem):
    c = lax.axis_index("c")
    s = lax.axis_index("s")
    wid = c * NS + s
    last = wid == NW - 1

    @pl.when(jnp.logical_not(last))
    def _():
        pltpu.async_copy(
            ei_hbm.at[:, pl.ds(wid * E_DEG, E_DEG)],
            idx_v.at[:, pl.ds(0, E_DEG)],
            dsem,
        )

    @pl.when(last)
    def _():
        pltpu.async_copy(
            ei_hbm.at[:, pl.ds((NW - 1) * E_DEG, E_LAST)], idx_v, dsem
        )

    zeros16 = jnp.zeros((16,), jnp.float32)

    def _zero(i, carry):
        hist_v[pl.ds(i * 16, 16)] = zeros16
        return carry

    lax.fori_loop(0, NHIST // 16, _zero, 0)

    @pl.when(jnp.logical_not(last))
    def _():
        pltpu.make_async_copy(
            ei_hbm.at[:, pl.ds(wid * E_DEG, E_DEG)],
            idx_v.at[:, pl.ds(0, E_DEG)],
            dsem,
        ).wait()

    @pl.when(last)
    def _():
        pltpu.make_async_copy(
            ei_hbm.at[:, pl.ds((NW - 1) * E_DEG, E_LAST)], idx_v, dsem
        ).wait()

    ones16 = jnp.ones((16,), jnp.float32)

    def _count(j, carry):
        ix = idx_v[0, pl.ds(j * 16, 16)]
        plsc.addupdate_scatter(hist_v, [ix], ones16)
        return carry

    nj = jnp.where(last, E_LAST // 16, E_DEG // 16)
    lax.fori_loop(0, nj, _count, 0)

    pltpu.sync_copy(hist_v, hist_sh.at[pl.ds(s * NHIST, NHIST)])
    plsc.subcore_barrier()
    for k in range(NS):
        pltpu.sync_copy(
            hist_sh.at[pl.ds(k * NHIST + s * CPW, CPW)],
            cols_v.at[pl.ds(k * CPW, CPW)],
        )

    def _sum(j, carry):
        val = cols_v[pl.ds(j * 16, 16)]
        for k in range(1, NS):
            val = val + cols_v[pl.ds(k * CPW + j * 16, 16)]
        acc_v[pl.ds(j * 16, 16)] = val
        return carry

    lax.fori_loop(0, CPW // 16, _sum, 0)
    pltpu.sync_copy(acc_v, hist_out.at[pl.ds(c * NHIST + s * CPW, CPW)])


def _prescale_body(x_ref, w_ref, hist_ref, hs_ref):
    i = pl.program_id(0)
    row = i * _PRE_BLK + lax.broadcasted_iota(jnp.int32, (_PRE_BLK, 1), 0)
    h0 = hist_ref[pl.ds(i * _PRE_BLK, _PRE_BLK)]
    h1 = hist_ref[pl.ds(NHIST + i * _PRE_BLK, _PRE_BLK)]
    deg = jnp.reshape(h0 + h1 + 1.0, (_PRE_BLK, 1))
    nrm = lax.rsqrt(deg)
    h = jnp.dot(x_ref[...], w_ref[...], preferred_element_type=jnp.float32)
    hs_ref[...] = jnp.where(row < N, h * nrm, 0.0)


_PRE_BLK = 1280


def _prescale(x, w, hist):
    return pl.pallas_call(
        _prescale_body,
        grid=(NHIST // _PRE_BLK,),
        in_specs=[
            pl.BlockSpec((_PRE_BLK, D), lambda i: (i, 0)),
            pl.BlockSpec((D, D), lambda i: (0, 0)),
            pl.BlockSpec((NC * NHIST,), lambda i: (0,)),
        ],
        out_specs=pl.BlockSpec((_PRE_BLK, D), lambda i: (i, 0)),
        out_shape=jax.ShapeDtypeStruct((NHIST, D), jnp.float32),
    )(x, w, hist)


@functools.partial(
    pl.kernel,
    out_type=jax.ShapeDtypeStruct((NC, NPAD, D), jnp.float32),
    mesh=_MESH,
    compiler_params=pltpu.CompilerParams(needs_layout_passes=False),
    scratch_types=[
        pltpu.VMEM((CPT // NPH, CH), jnp.int32),
        pltpu.VMEM((CPT // NPH, CH), jnp.int32),
        [pltpu.VMEM((CH, D), jnp.float32)] * NSLOT,
        pltpu.VMEM_SHARED((NPAD, D), jnp.float32),
        [pltpu.SemaphoreType.DMA] * NSLOT,
        [pltpu.SemaphoreType.DMA] * NSLOT,
        pltpu.SemaphoreType.DMA,
    ],
)
def _sc_aggregate(
    hs_hbm, edges_hbm, out_hbm, sidx, didx, rows, agg_sh, gsems, ssems, seedsem
):
    c = lax.axis_index("c")
    s = lax.axis_index("s")
    wid = c * NS + s
    base = s * RPT

    @pl.when(c == 0)
    def _():
        pltpu.async_copy(
            hs_hbm.at[pl.ds(base, RPT)], agg_sh.at[pl.ds(base, RPT)], seedsem
        )

    @pl.when(c != 0)
    def _():
        for k in range(5):
            pltpu.async_copy(
                hs_hbm.at[pl.ds(N, 112)],
                agg_sh.at[pl.ds(base + k * 112, 112)],
                seedsem,
            )
        pltpu.async_copy(
            hs_hbm.at[pl.ds(N, 72)], agg_sh.at[pl.ds(base + 560, 72)], seedsem
        )

    CPP = CPT // NPH
    for p in range(NPH):
        pltpu.sync_copy(edges_hbm.at[0, pl.ds(wid * CPT + p * CPP, CPP)], sidx)
        pltpu.sync_copy(edges_hbm.at[1, pl.ds(wid * CPT + p * CPP, CPP)], didx)

        for t in range(NSLOT):
            pltpu.async_copy(hs_hbm.at[sidx.at[t]], rows[t], gsems[t])

        if p == 0:
            @pl.when(c == 0)
            def _():
                pltpu.make_async_copy(
                    hs_hbm.at[pl.ds(base, RPT)],
                    agg_sh.at[pl.ds(base, RPT)],
                    seedsem,
                ).wait()

            @pl.when(c != 0)
            def _():
                for k in range(5):
                    pltpu.make_async_copy(
                        hs_hbm.at[pl.ds(N, 112)],
                        agg_sh.at[pl.ds(base + k * 112, 112)],
                        seedsem,
                    ).wait()
                pltpu.make_async_copy(
                    hs_hbm.at[pl.ds(N, 72)],
                    agg_sh.at[pl.ds(base + 560, 72)],
                    seedsem,
                ).wait()

            plsc.subcore_barrier()

        def _group(i, carry):
            for t in range(NSLOT):
                j = i * NSLOT + t
                pltpu.make_async_copy(
                    hs_hbm.at[sidx.at[j]], rows[t], gsems[t]
                ).wait()
                pltpu.async_copy(rows[t], agg_sh.at[didx.at[j]], ssems[t], add=True)

                @pl.when(i < CPP // NSLOT - 1)
                def _():
                    pltpu.make_async_copy(
                        rows[t], agg_sh.at[didx.at[j]], ssems[t]
                    ).wait()
                    pltpu.async_copy(
                        hs_hbm.at[sidx.at[j + NSLOT]], rows[t], gsems[t]
                    )

            return carry

        lax.fori_loop(0, CPP // NSLOT, _group, 0)

        for t in range(NSLOT):
            pltpu.make_async_copy(
                rows[t], agg_sh.at[didx.at[CPP - NSLOT + t]], ssems[t]
            ).wait()

    plsc.subcore_barrier()
    pltpu.sync_copy(
        agg_sh.at[pl.ds(base, RPT)], out_hbm.at[c, pl.ds(base, RPT)]
    )


def _final_body(a_ref, b_ref, out_ref):
    r = a_ref[0] + b_ref[0]
    ss = jnp.sum(r * r, axis=1, keepdims=True)
    denom = jnp.maximum(jnp.sqrt(ss), 1e-12)
    out_ref[...] = jnp.tanh(r / denom)


_FIN_BLK = 2000


def _final(partials):
    return pl.pallas_call(
        _final_body,
        grid=(N // _FIN_BLK,),
        in_specs=[
            pl.BlockSpec((1, _FIN_BLK, D), lambda i: (0, i, 0)),
            pl.BlockSpec((1, _FIN_BLK, D), lambda i: (1, i, 0)),
        ],
        out_specs=pl.BlockSpec((_FIN_BLK, D), lambda i: (i, 0)),
        out_shape=jax.ShapeDtypeStruct((N, D), jnp.float32),
    )(partials, partials)


@jax.jit
def kernel(x, edge_index, W):
    npe = EPAD - E
    pad_src = N + jnp.arange(npe, dtype=jnp.int32) % (NPAD - N)
    pad_dst = jnp.arange(npe, dtype=jnp.int32) % N
    pad2 = jnp.stack([pad_src, pad_dst]).reshape(2, npe // CH, CH)
    edges_p = jnp.concatenate(
        [edge_index.reshape(2, E // CH, CH), pad2], axis=1
    )

    hist = _sc_degree(edge_index)
    hs = _prescale(x, W, hist)
    partials = _sc_aggregate(hs, edges_p)
    return _final(partials)
